# Initial kernel scaffold; baseline (speedup 1.0000x reference)
import functools

import jax
import jax.numpy as jnp
from jax import lax
from jax.experimental import pallas as pl
from jax.experimental.pallas import tpu as pltpu
from jax.experimental.pallas import tpu_sc as plsc


def _emb_mean_kernel(num_bags, hist, emb_dim, num_workers):
    bags_per_worker = num_bags // num_workers
    pairs_per_worker = bags_per_worker // 2
    ids_per_pair = 2 * hist
    idx_minor = 100
    gathers_per_pair = ids_per_pair // idx_minor
    idx_rows_per_worker = pairs_per_worker * gathers_per_pair
    out_per_worker = bags_per_worker * emb_dim

    mesh = plsc.VectorSubcoreMesh(core_axis_name="c", subcore_axis_name="s")

    @functools.partial(
        pl.kernel,
        mesh=mesh,
        out_type=jax.ShapeDtypeStruct((num_bags * emb_dim,), jnp.float32),
        scratch_types=[
            pltpu.VMEM((2 * gathers_per_pair, idx_minor), jnp.int32),
            pltpu.VMEM((2, ids_per_pair, emb_dim), jnp.float32),
            pltpu.VMEM((out_per_worker,), jnp.float32),
            pltpu.SemaphoreType.DMA,
            pltpu.SemaphoreType.DMA,
        ],
    )
    def body(idx_hbm, table_hbm, out_hbm, idx_v, rows_v, out_v, sem0, sem1):
        nc = 2
        wid = lax.axis_index("s") * nc + lax.axis_index("c")
        idx_row_base = wid * idx_rows_per_worker
        out_base = wid * out_per_worker

        lane = lax.iota(jnp.int32, 16)
        col_idx = lax.bitwise_and(lane, 7)
        row_const = jnp.where(lane >= 8, hist, 0).astype(jnp.int32)
        sems = (sem0, sem1)

        def start_gathers(p, buf):
            pltpu.sync_copy(
                idx_hbm.at[pl.ds(idx_row_base + p * gathers_per_pair,
                                 gathers_per_pair)],
                idx_v.at[pl.ds(buf * gathers_per_pair, gathers_per_pair)],
            )
            for g in range(gathers_per_pair):
                pltpu.async_copy(
                    table_hbm.at[idx_v.at[buf * gathers_per_pair + g]],
                    rows_v.at[buf, pl.ds(g * idx_minor, idx_minor)],
                    sems[buf],
                )

        def wait_gathers(buf):
            pltpu.make_async_copy(
                table_hbm.at[pl.ds(0, ids_per_pair)],
                rows_v.at[buf],
                sems[buf],
            ).wait()

        def accum(buf):
            rows = rows_v.at[buf]

            def inner(j, acc):
                v = plsc.load_gather(rows, [row_const + j, col_idx])
                return acc + v

            return lax.fori_loop(0, hist, inner,
                                 jnp.zeros((16,), jnp.float32), unroll=8)

        start_gathers(0, 0)
        start_gathers(1, 1)

        def step(p, _):
            buf = lax.rem(p, 2)

            def do(b):
                wait_gathers(b)
                acc = accum(b)

                @pl.when(p < pairs_per_worker - 2)
                def _():
                    start_gathers(p + 2, b)

                out_v[pl.ds(p * 16, 16)] = acc * (1.0 / hist)

            @pl.when(buf == 0)
            def _():
                do(0)

            @pl.when(buf == 1)
            def _():
                do(1)

            return 0

        lax.fori_loop(0, pairs_per_worker, step, 0)
        pltpu.sync_copy(out_v, out_hbm.at[pl.ds(out_base, out_per_worker)])

    return body


def kernel(input_tensor, table):
    num_bags, hist = input_tensor.shape
    vocab, emb_dim = table.shape
    num_workers = 32
    idx_minor = 100
    idx = input_tensor.astype(jnp.int32).reshape(-1, idx_minor)
    k = _emb_mean_kernel(num_bags, hist, emb_dim, num_workers)
    out_flat = k(idx, table)
    return out_flat.reshape(num_bags, emb_dim)


# speedup vs baseline: 77.0136x; 77.0136x over previous
import functools

import jax
import jax.numpy as jnp
from jax import lax
from jax.experimental import pallas as pl
from jax.experimental.pallas import tpu as pltpu
from jax.experimental.pallas import tpu_sc as plsc


def _emb_mean_kernel(num_bags, hist, emb_dim, num_workers):
    bags_per_worker = num_bags // num_workers
    pairs_per_worker = bags_per_worker // 2
    ids_per_pair = 2 * hist
    idx_minor = 100
    gathers_per_pair = ids_per_pair // idx_minor
    idx_rows_per_worker = pairs_per_worker * gathers_per_pair
    out_per_worker = bags_per_worker * emb_dim

    mesh = plsc.VectorSubcoreMesh(core_axis_name="c", subcore_axis_name="s")

    @functools.partial(
        pl.kernel,
        mesh=mesh,
        out_type=jax.ShapeDtypeStruct((num_bags * emb_dim,), jnp.float32),
        compiler_params=pltpu.CompilerParams(
            needs_layout_passes=False, use_tc_tiling_on_sc=False),
        scratch_types=[
            pltpu.VMEM((2 * gathers_per_pair, idx_minor), jnp.int32),
            pltpu.VMEM((2, ids_per_pair, emb_dim), jnp.float32),
            pltpu.VMEM((out_per_worker,), jnp.float32),
            pltpu.SemaphoreType.DMA,
            pltpu.SemaphoreType.DMA,
        ],
    )
    def body(idx_hbm, table_hbm, out_hbm, idx_v, rows_v, out_v, sem0, sem1):
        nc = 2
        wid = lax.axis_index("s") * nc + lax.axis_index("c")
        idx_row_base = wid * idx_rows_per_worker
        out_base = wid * out_per_worker

        lane = lax.iota(jnp.int32, 16)
        col_idx = lax.bitwise_and(lane, 7)
        row_const = jnp.where(lane >= 8, hist, 0).astype(jnp.int32)
        sems = (sem0, sem1)

        def start_gathers(p, buf):
            pltpu.sync_copy(
                idx_hbm.at[pl.ds(idx_row_base + p * gathers_per_pair,
                                 gathers_per_pair)],
                idx_v.at[pl.ds(buf * gathers_per_pair, gathers_per_pair)],
            )
            for g in range(gathers_per_pair):
                pltpu.async_copy(
                    table_hbm.at[idx_v.at[buf * gathers_per_pair + g]],
                    rows_v.at[buf, pl.ds(g * idx_minor, idx_minor)],
                    sems[buf],
                )

        def wait_gathers(buf):
            pltpu.make_async_copy(
                table_hbm.at[pl.ds(0, ids_per_pair)],
                rows_v.at[buf],
                sems[buf],
            ).wait()

        def accum(buf):
            rows = rows_v.at[buf]

            def inner(j, acc):
                v = plsc.load_gather(rows, [row_const + j, col_idx])
                return acc + v

            return lax.fori_loop(0, hist, inner,
                                 jnp.zeros((16,), jnp.float32), unroll=8)

        start_gathers(0, 0)
        start_gathers(1, 1)

        def step(p, _):
            buf = lax.rem(p, 2)

            def do(b):
                wait_gathers(b)
                acc = accum(b)

                @pl.when(p < pairs_per_worker - 2)
                def _():
                    start_gathers(p + 2, b)

                out_v[pl.ds(p * 16, 16)] = acc * (1.0 / hist)

            @pl.when(buf == 0)
            def _():
                do(0)

            @pl.when(buf == 1)
            def _():
                do(1)

            return 0

        lax.fori_loop(0, pairs_per_worker, step, 0)
        pltpu.sync_copy(out_v, out_hbm.at[pl.ds(out_base, out_per_worker)])

    return body


def kernel(input_tensor, table):
    num_bags, hist = input_tensor.shape
    vocab, emb_dim = table.shape
    num_workers = 32
    idx_minor = 100
    idx = input_tensor.astype(jnp.int32).reshape(-1, idx_minor)
    k = _emb_mean_kernel(num_bags, hist, emb_dim, num_workers)
    out_flat = k(idx, table)
    return out_flat.reshape(num_bags, emb_dim)


# speedup vs baseline: 91.3970x; 1.1868x over previous
import functools

import jax
import jax.numpy as jnp
from jax import lax
from jax.experimental import pallas as pl
from jax.experimental.pallas import tpu as pltpu
from jax.experimental.pallas import tpu_sc as plsc

_NBUF = 4


def _emb_mean_kernel(num_bags, hist, emb_dim, num_workers):
    bags_per_worker = num_bags // num_workers
    pairs_per_worker = bags_per_worker // 2
    ids_per_pair = 2 * hist
    idx_minor = 100
    gathers_per_pair = ids_per_pair // idx_minor
    idx_rows_per_worker = pairs_per_worker * gathers_per_pair
    out_per_worker = bags_per_worker * emb_dim

    mesh = plsc.VectorSubcoreMesh(core_axis_name="c", subcore_axis_name="s")

    @functools.partial(
        pl.kernel,
        mesh=mesh,
        out_type=jax.ShapeDtypeStruct((num_bags * emb_dim,), jnp.float32),
        compiler_params=pltpu.CompilerParams(
            needs_layout_passes=False, use_tc_tiling_on_sc=False),
        scratch_types=[
            pltpu.VMEM((idx_rows_per_worker, idx_minor), jnp.int32),
            pltpu.VMEM((_NBUF * ids_per_pair, emb_dim), jnp.float32),
            pltpu.VMEM((out_per_worker,), jnp.float32),
            pltpu.SemaphoreType.DMA,
            pltpu.SemaphoreType.DMA,
            pltpu.SemaphoreType.DMA,
            pltpu.SemaphoreType.DMA,
        ],
    )
    def body(idx_hbm, table_hbm, out_hbm, idx_v, rows_v, out_v,
             sem0, sem1, sem2, sem3):
        nc = 2
        wid = lax.axis_index("s") * nc + lax.axis_index("c")
        out_base = wid * out_per_worker
        sems = (sem0, sem1, sem2, sem3)

        lane = lax.iota(jnp.int32, 16)
        col_idx = lax.bitwise_and(lane, 7)
        row_const = jnp.where(lane >= 8, hist, 0).astype(jnp.int32)
        ones = jnp.ones((16,), jnp.int32)

        pltpu.sync_copy(
            idx_hbm.at[pl.ds(wid * idx_rows_per_worker, idx_rows_per_worker)],
            idx_v,
        )

        def fire(p, b):
            for g in range(gathers_per_pair):
                pltpu.async_copy(
                    table_hbm.at[idx_v.at[p * gathers_per_pair + g]],
                    rows_v.at[pl.ds(b * ids_per_pair + g * idx_minor,
                                    idx_minor)],
                    sems[b],
                )

        def wait(p, b):
            pltpu.make_async_copy(
                table_hbm.at[pl.ds(0, ids_per_pair)],
                rows_v.at[pl.ds(b * ids_per_pair, ids_per_pair)],
                sems[b],
            ).wait()

        for b in range(_NBUF):
            fire(b, b)

        def step(p, _):
            parity = lax.bitwise_and(p, _NBUF - 1)
            base = parity * ids_per_pair

            for b in range(_NBUF):
                @pl.when(parity == b)
                def _(b=b):
                    wait(p, b)

            def inner(j, carry):
                acc, rowv = carry
                v = plsc.load_gather(rows_v, [rowv, col_idx])
                return acc + v, rowv + ones

            acc, _rowv = lax.fori_loop(
                0, hist, inner,
                (jnp.zeros((16,), jnp.float32), row_const + base),
                unroll=16)

            @pl.when(p < pairs_per_worker - _NBUF)
            def _():
                for b in range(_NBUF):
                    @pl.when(parity == b)
                    def _(b=b):
                        fire(p + _NBUF, b)

            out_v[pl.ds(p * 16, 16)] = acc * (1.0 / hist)
            return 0

        lax.fori_loop(0, pairs_per_worker, step, 0)
        pltpu.sync_copy(out_v, out_hbm.at[pl.ds(out_base, out_per_worker)])

    return body


def kernel(input_tensor, table):
    num_bags, hist = input_tensor.shape
    vocab, emb_dim = table.shape
    num_workers = 32
    idx_minor = 100
    idx = input_tensor.astype(jnp.int32).reshape(-1, idx_minor)
    k = _emb_mean_kernel(num_bags, hist, emb_dim, num_workers)
    out_flat = k(idx, table)
    return out_flat.reshape(num_bags, emb_dim)
